# baseline (speedup 1.0000x reference)
import jax
import jax.numpy as jnp
from jax import lax
from jax.experimental import pallas as pl
from jax.experimental.pallas import tpu as pltpu
from jax.experimental.pallas import tpu_sc as plsc

BATCH = 8192
SHAPE = 512
NUM_BINS = 10
TAIL = 10.0
MIN_W = 1e-3
MIN_H = 1e-3
MIN_D = 1e-3

NC = 2
NS = 16
L = 16
NW = NC * NS
ROWS_W = BATCH // NW
BLK = 16
NBLK = ROWS_W // BLK
NPAIR = NBLK // 2
NG = SHAPE // L

_LN2 = 0.6931471805599453
_SQRT2 = 1.4142135623730951


def _ln(x):
    bits = plsc.bitcast(x, jnp.int32)
    e = (bits >> 23) - 127
    m = plsc.bitcast((bits & 0x007FFFFF) | 0x3F800000, jnp.float32)
    big = m >= _SQRT2
    m = jnp.where(big, m * 0.5, m)
    e = jnp.where(big, e + 1, e)
    t = (m - 1.0) / (m + 1.0)
    u = t * t
    poly = 1.0 + u * (1.0 / 3.0 + u * (1.0 / 5.0 + u * (1.0 / 7.0)))
    return e.astype(jnp.float32) * _LN2 + (2.0 * t) * poly


def _softplus(x):
    return jnp.maximum(x, 0.0) + _ln(1.0 + jnp.exp(-jnp.abs(x)))


def _body(x_hbm, uw_hbm, uh_hbm, ud_hbm, out_hbm, ld_hbm,
          uw_v, uh_v, ud_v, cw_v, rw_v, ch_v, dl_v,
          A_v, B_v, C_v, E_v, F_v, G_v, S_v,
          in0_v, in1_v, out0_v, out1_v, ldp_v, ld_v,
          si0, si1, so0, so1):
    wid = lax.axis_index("s") * NC + lax.axis_index("c")
    base = wid * ROWS_W

    pltpu.sync_copy(uw_hbm, uw_v)
    pltpu.sync_copy(uh_hbm, uh_v)
    pltpu.sync_copy(ud_hbm, ud_v)

    iota = lax.iota(jnp.int32, L)

    def _norm_cum(vals):
        m = vals[0]
        for v in vals[1:]:
            m = jnp.maximum(m, v)
        es = [jnp.exp(v - m) for v in vals]
        s = es[0]
        for v in es[1:]:
            s = s + v
        rs = 1.0 / s
        knots = [jnp.full((L,), -TAIL, jnp.float32)]
        c = jnp.zeros((L,), jnp.float32)
        for k in range(NUM_BINS):
            w = MIN_W + (1.0 - MIN_W * NUM_BINS) * (es[k] * rs)
            c = c + w
            if k == NUM_BINS - 1:
                knots.append(jnp.full((L,), TAIL, jnp.float32))
            else:
                knots.append(2.0 * TAIL * c - TAIL)
        return knots

    big = jnp.full((L,), 3.0e38, jnp.float32)

    def _prep(g, _):
        c0 = g * L
        cols = iota + c0

        def par(ref, k, nb):
            return plsc.load_gather(ref, [cols * nb + k])

        cw = _norm_cum([par(uw_v, k, NUM_BINS) for k in range(NUM_BINS)])
        ch = _norm_cum([par(uh_v, k, NUM_BINS) for k in range(NUM_BINS)])
        for j in range(16):
            S_v[pl.ds(j * SHAPE + c0, L)] = cw[j + 1] if j < NUM_BINS - 1 else big
        one = jnp.full((L,), 1.0, jnp.float32)
        dd = ([one]
              + [MIN_D + _softplus(par(ud_v, k, NUM_BINS - 1))
                 for k in range(NUM_BINS - 1)]
              + [one])
        for k in range(NUM_BINS + 1):
            cw_v[pl.ds(k * SHAPE + c0, L)] = cw[k]
        for k in range(NUM_BINS):
            w = cw[k + 1] - cw[k]
            h = ch[k + 1] - ch[k]
            rw = 1.0 / w
            dl = h * rw
            d, dp = dd[k], dd[k + 1]
            dl2 = dl * dl
            Cc = d + dp - (dl + dl)
            sl = pl.ds(k * SHAPE + c0, L)
            rw_v[sl] = rw
            ch_v[sl] = ch[k]
            dl_v[sl] = dl
            A_v[sl] = h * (dl - d)
            B_v[sl] = h * d
            C_v[sl] = Cc
            E_v[sl] = dl2 * Cc
            F_v[sl] = (dl2 + dl2) * (dl - d)
            G_v[sl] = dl2 * d
        return 0

    lax.fori_loop(0, NG, _prep, 0)

    def _compute(bi, in_v, out_v):
        def _row(r, _):
            @plsc.parallel_loop(0, NG,
                                carry=(jnp.zeros((L,), jnp.int32),
                                       jnp.full((L,), 1.0, jnp.float32)),
                                unroll=3)
            def _group(g, carry):
                acc_e, acc_m = carry
                c0 = g * L
                x = in_v[r, pl.ds(c0, L)]
                xc = jnp.minimum(jnp.maximum(x, -TAIL), TAIL)
                idx = iota + c0
                p8 = S_v[pl.ds(7 * SHAPE + c0, L)]
                idx = idx + jnp.where(xc >= p8, 8 * SHAPE, 0)
                p4 = plsc.load_gather(S_v, [idx + (3 * SHAPE)])
                idx = idx + jnp.where(xc >= p4, 4 * SHAPE, 0)
                p2 = plsc.load_gather(S_v, [idx + SHAPE])
                idx = idx + jnp.where(xc >= p2, 2 * SHAPE, 0)
                p1 = plsc.load_gather(S_v, [idx])
                idx = idx + jnp.where(xc >= p1, SHAPE, 0)
                cwb = plsc.load_gather(cw_v, [idx])
                rwb = plsc.load_gather(rw_v, [idx])
                chb = plsc.load_gather(ch_v, [idx])
                dlb = plsc.load_gather(dl_v, [idx])
                Ab = plsc.load_gather(A_v, [idx])
                Bb = plsc.load_gather(B_v, [idx])
                Cb = plsc.load_gather(C_v, [idx])
                Eb = plsc.load_gather(E_v, [idx])
                Fb = plsc.load_gather(F_v, [idx])
                Gb = plsc.load_gather(G_v, [idx])

                th = (xc - cwb) * rwb
                th2 = th * th
                u = th - th2
                num = (Ab * th + Bb) * th
                den = Cb * u + dlb
                rden = 1.0 / den
                out_s = chb + num * rden
                dnum = (Eb * th + Fb) * th + Gb
                ratio = dnum * rden * rden

                inside = jnp.abs(x) <= TAIL
                out_v[r, pl.ds(c0, L)] = jnp.where(inside, out_s, x)
                bits = plsc.bitcast(ratio, jnp.int32)
                mb = plsc.bitcast((bits & 0x007FFFFF) | 0x3F800000,
                                  jnp.float32)
                acc_e = acc_e + jnp.where(inside, bits >> 23, 127)
                acc_m = acc_m * jnp.where(inside, mb, 1.0)
                return acc_e, acc_m

            acc_e, acc_m = _group
            ldp_v[pl.ds(r * L, L)] = _ln(acc_m) + (
                acc_e.astype(jnp.float32) - 127.0 * NG) * _LN2
            return 0

        lax.fori_loop(0, BLK, _row, 0)

        rows16 = iota << 4
        tot = plsc.load_gather(ldp_v, [rows16])
        for c in range(1, L):
            tot = tot + plsc.load_gather(ldp_v, [rows16 + c])
        ld_v[pl.ds(bi * BLK, L)] = tot

    def _in_slice(b):
        return x_hbm.at[pl.ds(base + b * BLK, BLK), :]

    def _out_slice(b):
        return out_hbm.at[pl.ds(base + b * BLK, BLK), :]

    pltpu.async_copy(_in_slice(0), in0_v, si0)
    pltpu.async_copy(_in_slice(1), in1_v, si1)

    def _pair(t, _):
        bA = t * 2
        bB = bA + 1
        pltpu.make_async_copy(_in_slice(bA), in0_v, si0).wait()

        @pl.when(t > 0)
        def _():
            pltpu.make_async_copy(out0_v, _out_slice(bA), so0).wait()

        _compute(bA, in0_v, out0_v)
        pltpu.async_copy(out0_v, _out_slice(bA), so0)

        @pl.when(t + 1 < NPAIR)
        def _():
            pltpu.async_copy(_in_slice(bA + 2), in0_v, si0)

        pltpu.make_async_copy(_in_slice(bB), in1_v, si1).wait()

        @pl.when(t > 0)
        def _():
            pltpu.make_async_copy(out1_v, _out_slice(bB), so1).wait()

        _compute(bB, in1_v, out1_v)
        pltpu.async_copy(out1_v, _out_slice(bB), so1)

        @pl.when(t + 1 < NPAIR)
        def _():
            pltpu.async_copy(_in_slice(bB + 2), in1_v, si1)

        return 0

    lax.fori_loop(0, NPAIR, _pair, 0)
    pltpu.make_async_copy(out0_v, _out_slice(NBLK - 2), so0).wait()
    pltpu.make_async_copy(out1_v, _out_slice(NBLK - 1), so1).wait()
    pltpu.sync_copy(ld_v, ld_hbm.at[pl.ds(base, ROWS_W)])


@jax.jit
def kernel(inputs, unnormalized_widths, unnormalized_heights,
           unnormalized_derivatives):
    mesh = plsc.VectorSubcoreMesh(core_axis_name="c", subcore_axis_name="s")
    f = pl.kernel(
        _body,
        out_type=(
            jax.ShapeDtypeStruct((BATCH, SHAPE), jnp.float32),
            jax.ShapeDtypeStruct((BATCH,), jnp.float32),
        ),
        mesh=mesh,
        compiler_params=pltpu.CompilerParams(needs_layout_passes=False),
        scratch_types=[
            pltpu.VMEM((SHAPE * NUM_BINS,), jnp.float32),
            pltpu.VMEM((SHAPE * NUM_BINS,), jnp.float32),
            pltpu.VMEM((SHAPE * (NUM_BINS - 1),), jnp.float32),
            pltpu.VMEM(((NUM_BINS + 1) * SHAPE,), jnp.float32),
            pltpu.VMEM((NUM_BINS * SHAPE,), jnp.float32),
            pltpu.VMEM((NUM_BINS * SHAPE,), jnp.float32),
            pltpu.VMEM((NUM_BINS * SHAPE,), jnp.float32),
            pltpu.VMEM((NUM_BINS * SHAPE,), jnp.float32),
            pltpu.VMEM((NUM_BINS * SHAPE,), jnp.float32),
            pltpu.VMEM((NUM_BINS * SHAPE,), jnp.float32),
            pltpu.VMEM((NUM_BINS * SHAPE,), jnp.float32),
            pltpu.VMEM((NUM_BINS * SHAPE,), jnp.float32),
            pltpu.VMEM((NUM_BINS * SHAPE,), jnp.float32),
            pltpu.VMEM((16 * SHAPE,), jnp.float32),
            pltpu.VMEM((BLK, SHAPE), jnp.float32),
            pltpu.VMEM((BLK, SHAPE), jnp.float32),
            pltpu.VMEM((BLK, SHAPE), jnp.float32),
            pltpu.VMEM((BLK, SHAPE), jnp.float32),
            pltpu.VMEM((BLK * L,), jnp.float32),
            pltpu.VMEM((ROWS_W,), jnp.float32),
            pltpu.SemaphoreType.DMA,
            pltpu.SemaphoreType.DMA,
            pltpu.SemaphoreType.DMA,
            pltpu.SemaphoreType.DMA,
        ],
    )
    return f(inputs,
             unnormalized_widths.reshape(-1),
             unnormalized_heights.reshape(-1),
             unnormalized_derivatives.reshape(-1))


# speedup vs baseline: 1.1559x; 1.1559x over previous
import jax
import jax.numpy as jnp
from jax import lax
from jax.experimental import pallas as pl
from jax.experimental.pallas import tpu as pltpu
from jax.experimental.pallas import tpu_sc as plsc

BATCH = 8192
SHAPE = 512
NUM_BINS = 10
TAIL = 10.0
MIN_W = 1e-3
MIN_H = 1e-3
MIN_D = 1e-3

NC = 2
NS = 16
L = 16
NW = NC * NS
ROWS_W = BATCH // NW
BLK = 16
NBLK = ROWS_W // BLK
NPAIR = NBLK // 2
NG = SHAPE // L

_LN2 = 0.6931471805599453
_SQRT2 = 1.4142135623730951


def _ln(x):
    bits = plsc.bitcast(x, jnp.int32)
    e = (bits >> 23) - 127
    m = plsc.bitcast((bits & 0x007FFFFF) | 0x3F800000, jnp.float32)
    big = m >= _SQRT2
    m = jnp.where(big, m * 0.5, m)
    e = jnp.where(big, e + 1, e)
    t = (m - 1.0) / (m + 1.0)
    u = t * t
    poly = 1.0 + u * (1.0 / 3.0 + u * (1.0 / 5.0 + u * (1.0 / 7.0)))
    return e.astype(jnp.float32) * _LN2 + (2.0 * t) * poly


def _softplus(x):
    return jnp.maximum(x, 0.0) + _ln(1.0 + jnp.exp(-jnp.abs(x)))


def _body(x_hbm, uw_hbm, uh_hbm, ud_hbm, out_hbm, ld_hbm,
          uw_v, uh_v, ud_v, cw_v, rw_v, ch_v, dl_v,
          A_v, B_v, C_v, E_v, F_v, G_v, S_v,
          in0_v, in1_v, out0_v, out1_v, ldp_v, ld_v,
          si0, si1, so0, so1):
    wid = lax.axis_index("s") * NC + lax.axis_index("c")
    base = wid * ROWS_W

    pltpu.sync_copy(uw_hbm, uw_v)
    pltpu.sync_copy(uh_hbm, uh_v)
    pltpu.sync_copy(ud_hbm, ud_v)

    iota = lax.iota(jnp.int32, L)

    def _norm_cum(vals):
        m = vals[0]
        for v in vals[1:]:
            m = jnp.maximum(m, v)
        es = [jnp.exp(v - m) for v in vals]
        s = es[0]
        for v in es[1:]:
            s = s + v
        rs = 1.0 / s
        knots = [jnp.full((L,), -TAIL, jnp.float32)]
        c = jnp.zeros((L,), jnp.float32)
        for k in range(NUM_BINS):
            w = MIN_W + (1.0 - MIN_W * NUM_BINS) * (es[k] * rs)
            c = c + w
            if k == NUM_BINS - 1:
                knots.append(jnp.full((L,), TAIL, jnp.float32))
            else:
                knots.append(2.0 * TAIL * c - TAIL)
        return knots

    big = jnp.full((L,), 3.0e38, jnp.float32)

    def _prep(g, _):
        c0 = g * L
        cols = iota + c0

        def par(ref, k, nb):
            return plsc.load_gather(ref, [cols * nb + k])

        cw = _norm_cum([par(uw_v, k, NUM_BINS) for k in range(NUM_BINS)])
        ch = _norm_cum([par(uh_v, k, NUM_BINS) for k in range(NUM_BINS)])
        for j in range(16):
            S_v[pl.ds(j * SHAPE + c0, L)] = cw[j + 1] if j < NUM_BINS - 1 else big
        one = jnp.full((L,), 1.0, jnp.float32)
        dd = ([one]
              + [MIN_D + _softplus(par(ud_v, k, NUM_BINS - 1))
                 for k in range(NUM_BINS - 1)]
              + [one])
        for k in range(NUM_BINS + 1):
            cw_v[pl.ds(k * SHAPE + c0, L)] = cw[k]
        for k in range(NUM_BINS):
            w = cw[k + 1] - cw[k]
            h = ch[k + 1] - ch[k]
            rw = 1.0 / w
            dl = h * rw
            d, dp = dd[k], dd[k + 1]
            dl2 = dl * dl
            Cc = d + dp - (dl + dl)
            sl = pl.ds(k * SHAPE + c0, L)
            rw_v[sl] = rw
            ch_v[sl] = ch[k]
            dl_v[sl] = dl
            A_v[sl] = h * (dl - d)
            B_v[sl] = h * d
            C_v[sl] = Cc
            E_v[sl] = dl2 * Cc
            F_v[sl] = (dl2 + dl2) * (dl - d)
            G_v[sl] = dl2 * d
        return 0

    lax.fori_loop(0, NG, _prep, 0)

    def _compute(bi, in_v, out_v):
        def _row(r, _):
            @plsc.parallel_loop(0, NG,
                                carry=(jnp.zeros((L,), jnp.int32),
                                       jnp.full((L,), 1.0, jnp.float32)),
                                unroll=2)
            def _group(g, carry):
                acc_e, acc_m = carry
                c0 = g * L
                x = in_v[r, pl.ds(c0, L)]
                xc = jnp.minimum(jnp.maximum(x, -TAIL), TAIL)
                idx = iota + c0
                p8 = S_v[pl.ds(7 * SHAPE + c0, L)]
                idx = idx + jnp.where(xc >= p8, 8 * SHAPE, 0)
                p4 = plsc.load_gather(S_v, [idx + (3 * SHAPE)])
                idx = idx + jnp.where(xc >= p4, 4 * SHAPE, 0)
                p2 = plsc.load_gather(S_v, [idx + SHAPE])
                idx = idx + jnp.where(xc >= p2, 2 * SHAPE, 0)
                p1 = plsc.load_gather(S_v, [idx])
                idx = idx + jnp.where(xc >= p1, SHAPE, 0)
                cwb = plsc.load_gather(cw_v, [idx])
                rwb = plsc.load_gather(rw_v, [idx])
                chb = plsc.load_gather(ch_v, [idx])
                dlb = plsc.load_gather(dl_v, [idx])
                Ab = plsc.load_gather(A_v, [idx])
                Bb = plsc.load_gather(B_v, [idx])
                Cb = plsc.load_gather(C_v, [idx])
                Eb = plsc.load_gather(E_v, [idx])
                Fb = plsc.load_gather(F_v, [idx])
                Gb = plsc.load_gather(G_v, [idx])

                th = (xc - cwb) * rwb
                th2 = th * th
                u = th - th2
                num = (Ab * th + Bb) * th
                den = Cb * u + dlb
                rden = 1.0 / den
                out_s = chb + num * rden
                dnum = (Eb * th + Fb) * th + Gb
                ratio = dnum * rden * rden

                inside = jnp.abs(x) <= TAIL
                out_v[r, pl.ds(c0, L)] = jnp.where(inside, out_s, x)
                bits = plsc.bitcast(ratio, jnp.int32)
                mb = plsc.bitcast((bits & 0x007FFFFF) | 0x3F800000,
                                  jnp.float32)
                acc_e = acc_e + jnp.where(inside, bits >> 23, 127)
                acc_m = acc_m * jnp.where(inside, mb, 1.0)
                return acc_e, acc_m

            acc_e, acc_m = _group
            ldp_v[pl.ds(r * L, L)] = _ln(acc_m) + (
                acc_e.astype(jnp.float32) - 127.0 * NG) * _LN2
            return 0

        lax.fori_loop(0, BLK, _row, 0)

        rows16 = iota << 4
        tot = plsc.load_gather(ldp_v, [rows16])
        for c in range(1, L):
            tot = tot + plsc.load_gather(ldp_v, [rows16 + c])
        ld_v[pl.ds(bi * BLK, L)] = tot

    def _in_slice(b):
        return x_hbm.at[pl.ds(base + b * BLK, BLK), :]

    def _out_slice(b):
        return out_hbm.at[pl.ds(base + b * BLK, BLK), :]

    pltpu.async_copy(_in_slice(0), in0_v, si0)
    pltpu.async_copy(_in_slice(1), in1_v, si1)

    def _pair(t, _):
        bA = t * 2
        bB = bA + 1
        pltpu.make_async_copy(_in_slice(bA), in0_v, si0).wait()

        @pl.when(t > 0)
        def _():
            pltpu.make_async_copy(out0_v, _out_slice(bA), so0).wait()

        _compute(bA, in0_v, out0_v)
        pltpu.async_copy(out0_v, _out_slice(bA), so0)

        @pl.when(t + 1 < NPAIR)
        def _():
            pltpu.async_copy(_in_slice(bA + 2), in0_v, si0)

        pltpu.make_async_copy(_in_slice(bB), in1_v, si1).wait()

        @pl.when(t > 0)
        def _():
            pltpu.make_async_copy(out1_v, _out_slice(bB), so1).wait()

        _compute(bB, in1_v, out1_v)
        pltpu.async_copy(out1_v, _out_slice(bB), so1)

        @pl.when(t + 1 < NPAIR)
        def _():
            pltpu.async_copy(_in_slice(bB + 2), in1_v, si1)

        return 0

    lax.fori_loop(0, NPAIR, _pair, 0)
    pltpu.make_async_copy(out0_v, _out_slice(NBLK - 2), so0).wait()
    pltpu.make_async_copy(out1_v, _out_slice(NBLK - 1), so1).wait()
    pltpu.sync_copy(ld_v, ld_hbm.at[pl.ds(base, ROWS_W)])


@jax.jit
def kernel(inputs, unnormalized_widths, unnormalized_heights,
           unnormalized_derivatives):
    mesh = plsc.VectorSubcoreMesh(core_axis_name="c", subcore_axis_name="s")
    f = pl.kernel(
        _body,
        out_type=(
            jax.ShapeDtypeStruct((BATCH, SHAPE), jnp.float32),
            jax.ShapeDtypeStruct((BATCH,), jnp.float32),
        ),
        mesh=mesh,
        compiler_params=pltpu.CompilerParams(needs_layout_passes=False),
        scratch_types=[
            pltpu.VMEM((SHAPE * NUM_BINS,), jnp.float32),
            pltpu.VMEM((SHAPE * NUM_BINS,), jnp.float32),
            pltpu.VMEM((SHAPE * (NUM_BINS - 1),), jnp.float32),
            pltpu.VMEM(((NUM_BINS + 1) * SHAPE,), jnp.float32),
            pltpu.VMEM((NUM_BINS * SHAPE,), jnp.float32),
            pltpu.VMEM((NUM_BINS * SHAPE,), jnp.float32),
            pltpu.VMEM((NUM_BINS * SHAPE,), jnp.float32),
            pltpu.VMEM((NUM_BINS * SHAPE,), jnp.float32),
            pltpu.VMEM((NUM_BINS * SHAPE,), jnp.float32),
            pltpu.VMEM((NUM_BINS * SHAPE,), jnp.float32),
            pltpu.VMEM((NUM_BINS * SHAPE,), jnp.float32),
            pltpu.VMEM((NUM_BINS * SHAPE,), jnp.float32),
            pltpu.VMEM((NUM_BINS * SHAPE,), jnp.float32),
            pltpu.VMEM((16 * SHAPE,), jnp.float32),
            pltpu.VMEM((BLK, SHAPE), jnp.float32),
            pltpu.VMEM((BLK, SHAPE), jnp.float32),
            pltpu.VMEM((BLK, SHAPE), jnp.float32),
            pltpu.VMEM((BLK, SHAPE), jnp.float32),
            pltpu.VMEM((BLK * L,), jnp.float32),
            pltpu.VMEM((ROWS_W,), jnp.float32),
            pltpu.SemaphoreType.DMA,
            pltpu.SemaphoreType.DMA,
            pltpu.SemaphoreType.DMA,
            pltpu.SemaphoreType.DMA,
        ],
    )
    return f(inputs,
             unnormalized_widths.reshape(-1),
             unnormalized_heights.reshape(-1),
             unnormalized_derivatives.reshape(-1))
